# Initial kernel scaffold; baseline (speedup 1.0000x reference)
import jax
import jax.numpy as jnp
from jax.experimental import pallas as pl


def kernel(support_vector, query_vector):
    raise NotImplementedError("write your pallas kernel here")


# speedup vs baseline: 1.0028x; 1.0028x over previous
import jax
import jax.numpy as jnp
from jax.experimental import pallas as pl

_N = 1000
_D = 1024
_K = 30
_REG = 0.001
_NPAD = 1024
_RB = 64


def _knn_kernel(x_ref, data_ref, x2_ref, y2_ref, nbr_ref):
    i = pl.program_id(0)
    x = x_ref[...].astype(jnp.bfloat16)
    data = data_ref[...].astype(jnp.bfloat16)
    x2 = x2_ref[...]
    y2 = y2_ref[...]
    xy = jax.lax.dot_general(x, data, (((1,), (1,)), ((), ())),
                             preferred_element_type=jnp.float32)
    dist = jnp.maximum((x2 + y2) - 2.0 * xy, 0.0)
    col = jax.lax.broadcasted_iota(jnp.int32, (_RB, _NPAD), 1)
    row_ids = i * _RB + jax.lax.broadcasted_iota(jnp.int32, (_RB, _NPAD), 0)
    inf = jnp.float32(jnp.inf)
    dist = jnp.where(col == row_ids, -inf, dist)
    dist = jnp.where(col >= _N, inf, dist)
    big = jnp.int32(2 ** 30)
    for t in range(_K + 1):
        m = jnp.min(dist, axis=1, keepdims=True)
        cand = jnp.where(dist <= m, col, big)
        amin = jnp.min(cand, axis=1, keepdims=True)
        if t > 0:
            nbr_ref[:, t - 1:t] = amin
        dist = jnp.where(col == amin, inf, dist)


def _knn(data_pad, x2_col, y2_row):
    return pl.pallas_call(
        _knn_kernel,
        grid=(_NPAD // _RB,),
        in_specs=[
            pl.BlockSpec((_RB, _D), lambda i: (i, 0)),
            pl.BlockSpec((_NPAD, _D), lambda i: (0, 0)),
            pl.BlockSpec((_RB, 1), lambda i: (i, 0)),
            pl.BlockSpec((1, _NPAD), lambda i: (0, 0)),
        ],
        out_specs=pl.BlockSpec((_RB, _K), lambda i: (i, 0)),
        out_shape=jax.ShapeDtypeStruct((_NPAD, _K), jnp.int32),
    )(data_pad, data_pad, x2_col, y2_row)


def _barycenter(X, indices):
    n = X.shape[0]
    k = indices.shape[1]
    eye_k = jnp.eye(k, dtype=X.dtype)

    def one(x, nbr):
        x_neighbors = X[nbr]
        res = x_neighbors - x[None, :]
        cov = res @ res.T
        tr = jnp.trace(cov)
        Rg = jnp.where(tr > 0, _REG * tr, _REG)
        cov = cov + Rg * eye_k
        w = jnp.sum(jnp.linalg.inv(cov), axis=1)
        return w / jnp.sum(w)

    W = jax.vmap(one)(X, indices)
    rows = jnp.arange(n)[:, None]
    return jnp.zeros((n, n), dtype=X.dtype).at[rows, indices].set(W)


def kernel(support_vector, query_vector):
    data = jnp.concatenate([support_vector, query_vector], axis=0)
    data_pad = jnp.pad(data, ((0, _NPAD - _N), (0, 0)))
    y2 = jnp.sum(data_pad * data_pad, axis=1)

    nbr = _knn(data_pad, y2[:, None], y2[None, :])[:_N]

    W = _barycenter(data, nbr)
    M = jnp.eye(_N, dtype=W.dtype) - W
    M = M.T @ M
    Dv, V = jnp.linalg.eigh(M)
    order = jnp.argsort(jnp.abs(Dv))
    pick = V[:, order[1:3]]
    s = support_vector.shape[0]
    return pick[:s], pick[s:]


# speedup vs baseline: 1.0029x; 1.0001x over previous
import jax
import jax.numpy as jnp
from jax.experimental import pallas as pl

_N = 1000
_D = 1024
_K = 30
_REG = 0.001
_NPAD = 1024
_RB = 64


_NET8 = [(0, 1), (2, 3), (4, 5), (6, 7),
         (0, 2), (1, 3), (4, 6), (5, 7),
         (1, 2), (5, 6),
         (0, 4), (1, 5), (2, 6), (3, 7),
         (2, 4), (3, 5),
         (1, 2), (3, 4), (5, 6)]
_NSEG = _NPAD // 128


def _knn_kernel(x_ref, data_ref, x2_ref, y2_ref, nbr_ref):
    i = pl.program_id(0)
    x = x_ref[...].astype(jnp.bfloat16)
    data = data_ref[...].astype(jnp.bfloat16)
    x2 = x2_ref[...]
    y2 = y2_ref[...]
    xy = jax.lax.dot_general(x, data, (((1,), (1,)), ((), ())),
                             preferred_element_type=jnp.float32)
    dist = jnp.maximum((x2 + y2) - 2.0 * xy, 0.0)
    col = jax.lax.broadcasted_iota(jnp.int32, (_RB, _NPAD), 1)
    row_ids = i * _RB + jax.lax.broadcasted_iota(jnp.int32, (_RB, _NPAD), 0)
    inf = jnp.float32(jnp.inf)
    dist = jnp.where(col == row_ids, -inf, dist)
    dist = jnp.where(col >= _N, inf, dist)

    vals = [dist[:, 128 * g:128 * (g + 1)] for g in range(_NSEG)]
    cols = [col[:, 128 * g:128 * (g + 1)] for g in range(_NSEG)]
    for a, b in _NET8:
        swap = (vals[b] < vals[a]) | ((vals[b] == vals[a]) & (cols[b] < cols[a]))
        va = jnp.where(swap, vals[b], vals[a])
        vb = jnp.where(swap, vals[a], vals[b])
        ca = jnp.where(swap, cols[b], cols[a])
        cb = jnp.where(swap, cols[a], cols[b])
        vals[a], vals[b], cols[a], cols[b] = va, vb, ca, cb

    big = jnp.int32(2 ** 30)
    for t in range(_K + 1):
        m = jnp.min(vals[0], axis=1, keepdims=True)
        cand = jnp.where(vals[0] <= m, cols[0], big)
        amin = jnp.min(cand, axis=1, keepdims=True)
        if t > 0:
            nbr_ref[:, t - 1:t] = amin
        mask = cols[0] == amin
        for d2 in range(_NSEG - 1):
            vals[d2] = jnp.where(mask, vals[d2 + 1], vals[d2])
            cols[d2] = jnp.where(mask, cols[d2 + 1], cols[d2])
        vals[_NSEG - 1] = jnp.where(mask, inf, vals[_NSEG - 1])
        cols[_NSEG - 1] = jnp.where(mask, big, cols[_NSEG - 1])


def _knn(data_pad, x2_col, y2_row):
    return pl.pallas_call(
        _knn_kernel,
        grid=(_NPAD // _RB,),
        in_specs=[
            pl.BlockSpec((_RB, _D), lambda i: (i, 0)),
            pl.BlockSpec((_NPAD, _D), lambda i: (0, 0)),
            pl.BlockSpec((_RB, 1), lambda i: (i, 0)),
            pl.BlockSpec((1, _NPAD), lambda i: (0, 0)),
        ],
        out_specs=pl.BlockSpec((_RB, _K), lambda i: (i, 0)),
        out_shape=jax.ShapeDtypeStruct((_NPAD, _K), jnp.int32),
    )(data_pad, data_pad, x2_col, y2_row)


def _barycenter(X, indices):
    n = X.shape[0]
    k = indices.shape[1]
    eye_k = jnp.eye(k, dtype=X.dtype)

    def one(x, nbr):
        x_neighbors = X[nbr]
        res = x_neighbors - x[None, :]
        cov = res @ res.T
        tr = jnp.trace(cov)
        Rg = jnp.where(tr > 0, _REG * tr, _REG)
        cov = cov + Rg * eye_k
        w = jnp.sum(jnp.linalg.inv(cov), axis=1)
        return w / jnp.sum(w)

    W = jax.vmap(one)(X, indices)
    rows = jnp.arange(n)[:, None]
    return jnp.zeros((n, n), dtype=X.dtype).at[rows, indices].set(W)


def kernel(support_vector, query_vector):
    data = jnp.concatenate([support_vector, query_vector], axis=0)
    data_pad = jnp.pad(data, ((0, _NPAD - _N), (0, 0)))
    y2 = jnp.sum(data_pad * data_pad, axis=1)

    nbr = _knn(data_pad, y2[:, None], y2[None, :])[:_N]

    W = _barycenter(data, nbr)
    M = jnp.eye(_N, dtype=W.dtype) - W
    M = M.T @ M
    Dv, V = jnp.linalg.eigh(M)
    order = jnp.argsort(jnp.abs(Dv))
    pick = V[:, order[1:3]]
    s = support_vector.shape[0]
    return pick[:s], pick[s:]
